# baseline (device time: 55398 ns/iter reference)
import jax
import jax.numpy as jnp
from jax import lax
from jax.experimental import pallas as pl
from jax.experimental.pallas import tpu as pltpu

M = 2048
HALF = 1024
D = 1024


def kernel(partial, gamma):
    gamma2d = gamma.reshape(1, D)

    def body(p_ref, g_ref, out_ref, recv_buf, send_sem, recv_sem):
        my_x = lax.axis_index("x")
        my_y = lax.axis_index("y")
        nbr = (my_x, 1 - my_y)

        barrier_sem = pltpu.get_barrier_semaphore()
        pl.semaphore_signal(
            barrier_sem, inc=1, device_id=nbr,
            device_id_type=pl.DeviceIdType.MESH,
        )
        pl.semaphore_wait(barrier_sem, 1)

        other_start = (1 - my_y) * HALF
        rdma = pltpu.make_async_remote_copy(
            src_ref=p_ref.at[0, pl.ds(other_start, HALF), :],
            dst_ref=recv_buf,
            send_sem=send_sem,
            recv_sem=recv_sem,
            device_id=nbr,
            device_id_type=pl.DeviceIdType.MESH,
        )
        rdma.start()
        rdma.wait()

        mine_start = my_y * HALF
        y = p_ref[0, pl.ds(mine_start, HALF), :] + recv_buf[...]
        ms = jnp.sum(y * y, axis=-1, keepdims=True) * (1.0 / D)
        out_ref[...] = y * lax.rsqrt(ms + 1e-6) * g_ref[...]

    return pl.pallas_call(
        body,
        out_shape=jax.ShapeDtypeStruct((HALF, D), jnp.float32),
        in_specs=[
            pl.BlockSpec(memory_space=pltpu.VMEM),
            pl.BlockSpec(memory_space=pltpu.VMEM),
        ],
        out_specs=pl.BlockSpec(memory_space=pltpu.VMEM),
        scratch_shapes=[
            pltpu.VMEM((HALF, D), jnp.float32),
            pltpu.SemaphoreType.DMA,
            pltpu.SemaphoreType.DMA,
        ],
        compiler_params=pltpu.CompilerParams(collective_id=0),
    )(partial, gamma2d)


# device time: 37550 ns/iter; 1.4753x vs baseline; 1.4753x over previous
import jax
import jax.numpy as jnp
from jax import lax
from jax.experimental import pallas as pl
from jax.experimental.pallas import tpu as pltpu

M = 2048
HALF = 1024
XHALF = 512
D = 1024
K = 8
CR = XHALF // K


def kernel(partial, gamma):
    gamma2d = gamma.reshape(1, D)

    def body(p_ref, g_ref, out_ref, ybuf, xbuf, ysend, yrecv, xsend, xrecv):
        my_x = lax.axis_index("x")
        my_y = lax.axis_index("y")
        peer_y = (my_x, 1 - my_y)
        peer_x = (1 - my_x, my_y)

        barrier_sem = pltpu.get_barrier_semaphore()
        for peer in (peer_y, peer_x):
            pl.semaphore_signal(
                barrier_sem, inc=1, device_id=peer,
                device_id_type=pl.DeviceIdType.MESH,
            )
        pl.semaphore_wait(barrier_sem, 2)

        ysrc0 = (1 - my_y) * HALF + my_x * XHALF
        y_rdmas = []
        for c in range(K):
            r = pltpu.make_async_remote_copy(
                src_ref=p_ref.at[0, pl.ds(ysrc0 + c * CR, CR), :],
                dst_ref=ybuf.at[c],
                send_sem=ysend.at[c],
                recv_sem=yrecv.at[c],
                device_id=peer_y,
                device_id_type=pl.DeviceIdType.MESH,
            )
            r.start()
            y_rdmas.append(r)

        mine0 = my_y * HALF
        direct0 = my_x * XHALF
        fwd0 = (1 - my_x) * XHALF

        def rmsnorm_store(out_rows, local_rows, recv_chunk):
            y = p_ref[0, pl.ds(local_rows, CR), :] + recv_chunk
            ms = jnp.sum(y * y, axis=-1, keepdims=True) * (1.0 / D)
            out_ref[pl.ds(out_rows, CR), :] = y * lax.rsqrt(ms + 1e-6) * g_ref[...]

        x_rdmas = []
        for c in range(K):
            y_rdmas[c].wait_recv()
            r = pltpu.make_async_remote_copy(
                src_ref=ybuf.at[c],
                dst_ref=xbuf.at[c],
                send_sem=xsend.at[c],
                recv_sem=xrecv.at[c],
                device_id=peer_x,
                device_id_type=pl.DeviceIdType.MESH,
            )
            r.start()
            x_rdmas.append(r)
            rmsnorm_store(direct0 + c * CR, mine0 + direct0 + c * CR, ybuf[c])

        for c in range(K):
            x_rdmas[c].wait_recv()
            rmsnorm_store(fwd0 + c * CR, mine0 + fwd0 + c * CR, xbuf[c])

        for c in range(K):
            y_rdmas[c].wait_send()
            x_rdmas[c].wait_send()

    return pl.pallas_call(
        body,
        out_shape=jax.ShapeDtypeStruct((HALF, D), jnp.float32),
        in_specs=[
            pl.BlockSpec(memory_space=pltpu.VMEM),
            pl.BlockSpec(memory_space=pltpu.VMEM),
        ],
        out_specs=pl.BlockSpec(memory_space=pltpu.VMEM),
        scratch_shapes=[
            pltpu.VMEM((K, CR, D), jnp.float32),
            pltpu.VMEM((K, CR, D), jnp.float32),
            pltpu.SemaphoreType.DMA((K,)),
            pltpu.SemaphoreType.DMA((K,)),
            pltpu.SemaphoreType.DMA((K,)),
            pltpu.SemaphoreType.DMA((K,)),
        ],
        compiler_params=pltpu.CompilerParams(collective_id=0),
    )(partial, gamma2d)


# device time: 36715 ns/iter; 1.5089x vs baseline; 1.0227x over previous
import jax
import jax.numpy as jnp
from jax import lax
from jax.experimental import pallas as pl
from jax.experimental.pallas import tpu as pltpu

M = 2048
HALF = 1024
XHALF = 512
D = 1024
K = 16
CR = XHALF // K


def kernel(partial, gamma):
    gamma2d = gamma.reshape(1, D)

    def body(p_ref, g_ref, out_ref, ybuf, xbuf, ysend, yrecv, xsend, xrecv):
        my_x = lax.axis_index("x")
        my_y = lax.axis_index("y")
        peer_y = (my_x, 1 - my_y)
        peer_x = (1 - my_x, my_y)

        barrier_sem = pltpu.get_barrier_semaphore()
        for peer in (peer_y, peer_x):
            pl.semaphore_signal(
                barrier_sem, inc=1, device_id=peer,
                device_id_type=pl.DeviceIdType.MESH,
            )
        pl.semaphore_wait(barrier_sem, 2)

        ysrc0 = (1 - my_y) * HALF + my_x * XHALF
        y_rdmas = []
        for c in range(K):
            r = pltpu.make_async_remote_copy(
                src_ref=p_ref.at[0, pl.ds(ysrc0 + c * CR, CR), :],
                dst_ref=ybuf.at[c],
                send_sem=ysend.at[c],
                recv_sem=yrecv.at[c],
                device_id=peer_y,
                device_id_type=pl.DeviceIdType.MESH,
            )
            r.start()
            y_rdmas.append(r)

        mine0 = my_y * HALF
        direct0 = my_x * XHALF
        fwd0 = (1 - my_x) * XHALF

        def rmsnorm_store(out_rows, local_rows, recv_chunk):
            y = p_ref[0, pl.ds(local_rows, CR), :] + recv_chunk
            ms = jnp.sum(y * y, axis=-1, keepdims=True) * (1.0 / D)
            out_ref[pl.ds(out_rows, CR), :] = y * lax.rsqrt(ms + 1e-6) * g_ref[...]

        x_rdmas = []
        for c in range(K):
            y_rdmas[c].wait_recv()
            r = pltpu.make_async_remote_copy(
                src_ref=ybuf.at[c],
                dst_ref=xbuf.at[c],
                send_sem=xsend.at[c],
                recv_sem=xrecv.at[c],
                device_id=peer_x,
                device_id_type=pl.DeviceIdType.MESH,
            )
            r.start()
            x_rdmas.append(r)
            rmsnorm_store(direct0 + c * CR, mine0 + direct0 + c * CR, ybuf[c])

        for c in range(K):
            x_rdmas[c].wait_recv()
            rmsnorm_store(fwd0 + c * CR, mine0 + fwd0 + c * CR, xbuf[c])

        for c in range(K):
            y_rdmas[c].wait_send()
            x_rdmas[c].wait_send()

    return pl.pallas_call(
        body,
        out_shape=jax.ShapeDtypeStruct((HALF, D), jnp.float32),
        in_specs=[
            pl.BlockSpec(memory_space=pltpu.VMEM),
            pl.BlockSpec(memory_space=pltpu.VMEM),
        ],
        out_specs=pl.BlockSpec(memory_space=pltpu.VMEM),
        scratch_shapes=[
            pltpu.VMEM((K, CR, D), jnp.float32),
            pltpu.VMEM((K, CR, D), jnp.float32),
            pltpu.SemaphoreType.DMA((K,)),
            pltpu.SemaphoreType.DMA((K,)),
            pltpu.SemaphoreType.DMA((K,)),
            pltpu.SemaphoreType.DMA((K,)),
        ],
        compiler_params=pltpu.CompilerParams(collective_id=0),
    )(partial, gamma2d)
